# baseline (speedup 1.0000x reference)
import functools
import numpy as np
import jax
import jax.numpy as jnp
from jax import lax
from jax.experimental import pallas as pl
from jax.experimental.pallas import tpu as pltpu

_BT = 8
_G = 64


def _fused_kernel(x_ref, q_ref, p_ref, wq_ref, wk_ref, gw_ref, gb_ref,
                  bias_ref, r2n_ref, attn_ref, pm_ref, *, scale, k_order,
                  bt, r_dim, rp, n_dim, npad, g):
    bf = jnp.bfloat16
    f32 = jnp.float32
    qn = q_ref[...]
    gb = gb_ref[...]
    bias = bias_ref[...]
    gr = range(g)

    pbf = p_ref[...].astype(bf)
    pm_ref[...] = jnp.zeros((bt * rp, bt * npad), bf)
    for b in range(bt):
        pm_ref[b * rp:b * rp + r_dim, b * npad:b * npad + n_dim] = pbf
    pm = pm_ref[...]

    m_mat = lax.dot_general(wq_ref[...].astype(bf), wk_ref[...].astype(bf),
                            (((1,), (1,)), ((), ())),
                            preferred_element_type=f32).astype(bf)
    g0 = gw_ref[0].astype(bf)

    zrow = jnp.zeros((npad - n_dim, qn.shape[1]), bf)
    xfm = [jnp.concatenate(
             [v for b in range(bt)
              for v in (jnp.maximum(qn * x_ref[j * bt + b], 0.0).astype(bf),
                        zrow)], axis=0) for j in gr]
    xr = [jnp.dot(pm, xfm[j], preferred_element_type=f32) for j in gr]
    xrb = [v.astype(bf) for v in xr]

    xrm = [jnp.dot(xrb[j], m_mat, preferred_element_type=f32).astype(bf)
           for j in gr]
    dots = [lax.dot_general(xrm[j], xrb[j], (((1,), (1,)), ((), ())),
                            preferred_element_type=f32) for j in gr]
    if scale != 1.0:
        dots = [d * scale for d in dots]
    dots = [d + bias for d in dots]
    mx = [jnp.max(d, axis=-1, keepdims=True) for d in dots]
    ex = [jnp.exp(dots[j] - mx[j]) for j in gr]
    attn = [ex[j] * pl.reciprocal(jnp.sum(ex[j], axis=-1, keepdims=True),
                                  approx=True) for j in gr]

    for j in gr:
        for b in range(bt):
            attn_ref[j * bt + b] = attn[j][b * rp:b * rp + r_dim,
                                          b * rp:b * rp + r_dim]

    attnb = [a.astype(bf) for a in attn]
    h = xrb
    out = [jnp.dot(h[j], g0, preferred_element_type=f32) for j in gr]
    for kk in range(1, k_order):
        gk = gw_ref[kk].astype(bf)
        hg = [jnp.dot(h[j], gk, preferred_element_type=f32).astype(bf)
              for j in gr]
        out = [out[j] + jnp.dot(attnb[j], hg[j], preferred_element_type=f32)
               for j in gr]
        if kk + 1 < k_order:
            h = [jnp.dot(attnb[j], h[j],
                         preferred_element_type=f32).astype(bf) for j in gr]
    outb = [jnp.maximum(out[j] + gb, 0.0).astype(bf) for j in gr]

    for j in gr:
        r2n = lax.dot_general(pm, outb[j], (((0,), (0,)), ((), ())),
                              preferred_element_type=f32)
        for b in range(bt):
            r2n_ref[j * bt + b] = r2n[b * npad:b * npad + n_dim]


def kernel(x, Q, P, WqT, WkT, Wgcn, bgcn):
    B, N, D = x.shape
    R = P.shape[0]
    K, _, reg_dim = Wgcn.shape

    bt = _BT if B % _BT == 0 else 1
    g = next((gg for gg in (_G, 32, 16, 8, 4, 2, 1) if B % (bt * gg) == 0), 1)
    S = B // (bt * g)
    rp = -(-R // 8) * 8
    npad = -(-N // 8) * 8

    idx = np.arange(bt * rp)
    same_blk = (idx[:, None] // rp) == (idx[None, :] // rp)
    valid = ((idx[:, None] % rp) < R) & ((idx[None, :] % rp) < R)
    bias = jnp.asarray(np.where(same_blk & valid, 0.0, -1e30)
                       .astype(np.float32))

    kernel_fn = functools.partial(_fused_kernel, scale=1.0, k_order=K,
                                  bt=bt, r_dim=R, rp=rp, n_dim=N,
                                  npad=npad, g=g)

    out_shapes = (
        jax.ShapeDtypeStruct((B, N, reg_dim), jnp.float32),
        jax.ShapeDtypeStruct((B, R, R), jnp.float32),
    )

    grid_spec = pltpu.PrefetchScalarGridSpec(
        num_scalar_prefetch=0,
        grid=(S,),
        in_specs=[
            pl.BlockSpec((g * bt, N, D), lambda i: (i, 0, 0)),
            pl.BlockSpec((N, D), lambda i: (0, 0)),
            pl.BlockSpec((R, N), lambda i: (0, 0)),
            pl.BlockSpec((D, WqT.shape[1]), lambda i: (0, 0)),
            pl.BlockSpec((D, WkT.shape[1]), lambda i: (0, 0)),
            pl.BlockSpec((K, D, reg_dim), lambda i: (0, 0, 0)),
            pl.BlockSpec((1, reg_dim), lambda i: (0, 0)),
            pl.BlockSpec((bt * rp, bt * rp), lambda i: (0, 0)),
        ],
        out_specs=[
            pl.BlockSpec((g * bt, N, reg_dim), lambda i: (i, 0, 0)),
            pl.BlockSpec((g * bt, R, R), lambda i: (i, 0, 0)),
        ],
        scratch_shapes=[pltpu.VMEM((bt * rp, bt * npad), jnp.bfloat16)],
    )

    reg2node, A_reg = pl.pallas_call(
        kernel_fn,
        grid_spec=grid_spec,
        out_shape=out_shapes,
        compiler_params=pltpu.CompilerParams(
            dimension_semantics=("parallel",)),
    )(x, Q, P, WqT, WkT, Wgcn, bgcn, bias)

    return reg2node, A_reg


# speedup vs baseline: 1.0038x; 1.0038x over previous
import functools
import numpy as np
import jax
import jax.numpy as jnp
from jax import lax
from jax.experimental import pallas as pl
from jax.experimental.pallas import tpu as pltpu

_BT = 8
_G = 32


def _fused_kernel(x_ref, q_ref, p_ref, wq_ref, wk_ref, gw_ref, gb_ref,
                  bias_ref, r2n_ref, attn_ref, pm_ref, *, scale, k_order,
                  bt, r_dim, rp, n_dim, npad, g):
    bf = jnp.bfloat16
    f32 = jnp.float32
    qn = q_ref[...]
    gb = gb_ref[...]
    bias = bias_ref[...]
    gr = range(g)

    pbf = p_ref[...].astype(bf)
    pm_ref[...] = jnp.zeros((bt * rp, bt * npad), bf)
    for b in range(bt):
        pm_ref[b * rp:b * rp + r_dim, b * npad:b * npad + n_dim] = pbf
    pm = pm_ref[...]

    m_mat = lax.dot_general(wq_ref[...].astype(bf), wk_ref[...].astype(bf),
                            (((1,), (1,)), ((), ())),
                            preferred_element_type=f32).astype(bf)
    g0 = gw_ref[0].astype(bf)

    zrow = jnp.zeros((npad - n_dim, qn.shape[1]), bf)
    xfm = [jnp.concatenate(
             [v for b in range(bt)
              for v in (jnp.maximum(qn * x_ref[j * bt + b], 0.0).astype(bf),
                        zrow)], axis=0) for j in gr]
    xr = [jnp.dot(pm, xfm[j], preferred_element_type=f32) for j in gr]
    xrb = [v.astype(bf) for v in xr]

    xrm = [jnp.dot(xrb[j], m_mat, preferred_element_type=f32).astype(bf)
           for j in gr]
    dots = [lax.dot_general(xrm[j], xrb[j], (((1,), (1,)), ((), ())),
                            preferred_element_type=f32) for j in gr]
    if scale != 1.0:
        dots = [d * scale for d in dots]
    dots = [d + bias for d in dots]
    mx = [jnp.max(d, axis=-1, keepdims=True) for d in dots]
    ex = [jnp.exp(dots[j] - mx[j]) for j in gr]
    attn = [ex[j] * pl.reciprocal(jnp.sum(ex[j], axis=-1, keepdims=True),
                                  approx=True) for j in gr]

    for j in gr:
        for b in range(bt):
            attn_ref[j * bt + b] = attn[j][b * rp:b * rp + r_dim,
                                          b * rp:b * rp + r_dim]

    attnb = [a.astype(bf) for a in attn]
    h = xrb
    out = [jnp.dot(h[j], g0, preferred_element_type=f32) for j in gr]
    for kk in range(1, k_order):
        gk = gw_ref[kk].astype(bf)
        hg = [jnp.dot(h[j], gk, preferred_element_type=f32).astype(bf)
              for j in gr]
        out = [out[j] + jnp.dot(attnb[j], hg[j], preferred_element_type=f32)
               for j in gr]
        if kk + 1 < k_order:
            h = [jnp.dot(attnb[j], h[j],
                         preferred_element_type=f32).astype(bf) for j in gr]
    outb = [jnp.maximum(out[j] + gb, 0.0).astype(bf) for j in gr]

    for j in gr:
        r2n = lax.dot_general(pm, outb[j], (((0,), (0,)), ((), ())),
                              preferred_element_type=f32)
        for b in range(bt):
            r2n_ref[j * bt + b] = r2n[b * npad:b * npad + n_dim]


def kernel(x, Q, P, WqT, WkT, Wgcn, bgcn):
    B, N, D = x.shape
    R = P.shape[0]
    K, _, reg_dim = Wgcn.shape

    bt = _BT if B % _BT == 0 else 1
    g = next((gg for gg in (_G, 16, 8, 4, 2, 1) if B % (bt * gg) == 0), 1)
    S = B // (bt * g)
    rp = -(-R // 8) * 8
    npad = -(-N // 8) * 8

    idx = np.arange(bt * rp)
    same_blk = (idx[:, None] // rp) == (idx[None, :] // rp)
    valid = ((idx[:, None] % rp) < R) & ((idx[None, :] % rp) < R)
    bias = jnp.asarray(np.where(same_blk & valid, 0.0, -1e30)
                       .astype(np.float32))

    kernel_fn = functools.partial(_fused_kernel, scale=1.0, k_order=K,
                                  bt=bt, r_dim=R, rp=rp, n_dim=N,
                                  npad=npad, g=g)

    out_shapes = (
        jax.ShapeDtypeStruct((B, N, reg_dim), jnp.float32),
        jax.ShapeDtypeStruct((B, R, R), jnp.float32),
    )

    grid_spec = pltpu.PrefetchScalarGridSpec(
        num_scalar_prefetch=0,
        grid=(S,),
        in_specs=[
            pl.BlockSpec((g * bt, N, D), lambda i: (i, 0, 0)),
            pl.BlockSpec((N, D), lambda i: (0, 0)),
            pl.BlockSpec((R, N), lambda i: (0, 0)),
            pl.BlockSpec((D, WqT.shape[1]), lambda i: (0, 0)),
            pl.BlockSpec((D, WkT.shape[1]), lambda i: (0, 0)),
            pl.BlockSpec((K, D, reg_dim), lambda i: (0, 0, 0)),
            pl.BlockSpec((1, reg_dim), lambda i: (0, 0)),
            pl.BlockSpec((bt * rp, bt * rp), lambda i: (0, 0)),
        ],
        out_specs=[
            pl.BlockSpec((g * bt, N, reg_dim), lambda i: (i, 0, 0)),
            pl.BlockSpec((g * bt, R, R), lambda i: (i, 0, 0)),
        ],
        scratch_shapes=[pltpu.VMEM((bt * rp, bt * npad), jnp.bfloat16)],
    )

    reg2node, A_reg = pl.pallas_call(
        kernel_fn,
        grid_spec=grid_spec,
        out_shape=out_shapes,
        compiler_params=pltpu.CompilerParams(
            dimension_semantics=("arbitrary",)),
    )(x, Q, P, WqT, WkT, Wgcn, bgcn, bias)

    return reg2node, A_reg
